# Initial kernel scaffold; baseline (speedup 1.0000x reference)
import jax
import jax.numpy as jnp
from jax.experimental import pallas as pl


def kernel(x, mask, W, b):
    raise NotImplementedError("write your pallas kernel here")


# speedup vs baseline: 91.8046x; 91.8046x over previous
import functools

import jax
import jax.numpy as jnp
from jax import lax
from jax.experimental import pallas as pl
from jax.experimental.pallas import tpu as pltpu
from jax.experimental.pallas import tpu_sc as plsc


def _prep_body(x_ref, m_ref, qf_ref, sf_ref, sft_ref):
    xb = x_ref[0]
    mb = m_ref[0]
    qfb = xb * mb
    sfb = xb * (1.0 - mb)
    qf_ref[0] = qfb
    sf_ref[0] = sfb
    C, N = xb.shape
    sft_ref[0, :, :C] = jnp.transpose(sfb)
    sft_ref[0, :, C:] = jnp.zeros((N, 128 - C), jnp.float32)


def _prep(x3, m3):
    B, C, N = x3.shape
    return pl.pallas_call(
        _prep_body,
        grid=(B,),
        in_specs=[
            pl.BlockSpec((1, C, N), lambda b: (b, 0, 0)),
            pl.BlockSpec((1, 1, N), lambda b: (b, 0, 0)),
        ],
        out_specs=[
            pl.BlockSpec((1, C, N), lambda b: (b, 0, 0)),
            pl.BlockSpec((1, C, N), lambda b: (b, 0, 0)),
            pl.BlockSpec((1, N, 128), lambda b: (b, 0, 0)),
        ],
        out_shape=[
            jax.ShapeDtypeStruct((B, C, N), jnp.float32),
            jax.ShapeDtypeStruct((B, C, N), jnp.float32),
            jax.ShapeDtypeStruct((B, N, 128), jnp.float32),
        ],
    )(x3, m3)


def _top1_body(qf_ref, sf_ref, val_ref, idx_ref, *, n_total, blk):
    b = pl.program_id(0)
    qb = qf_ref[0]
    sb = sf_ref[0]
    s = lax.dot_general(qb, sb, (((0,), (0,)), ((), ())),
                        preferred_element_type=jnp.float32)
    mx = jnp.max(s, axis=1, keepdims=True)
    col = lax.broadcasted_iota(jnp.int32, s.shape, 1)
    am = jnp.min(jnp.where(s >= mx, col, n_total), axis=1)
    val_ref[...] = mx[:, 0].reshape(1, 1, blk)
    idx_ref[...] = (am + b * n_total).reshape(1, 1, blk)


def _top1(qf, sf, blk):
    B, C, N = qf.shape
    nb = N // blk
    body = functools.partial(_top1_body, n_total=N, blk=blk)
    return pl.pallas_call(
        body,
        grid=(B, nb),
        in_specs=[
            pl.BlockSpec((1, C, blk), lambda b, i: (b, 0, i)),
            pl.BlockSpec((1, C, N), lambda b, i: (b, 0, 0)),
        ],
        out_specs=[
            pl.BlockSpec((1, 1, blk), lambda b, i: (b * nb + i, 0, 0)),
            pl.BlockSpec((1, 1, blk), lambda b, i: (b * nb + i, 0, 0)),
        ],
        out_shape=[
            jax.ShapeDtypeStruct((B * nb, 1, blk), jnp.float32),
            jax.ShapeDtypeStruct((B * nb, 1, blk), jnp.int32),
        ],
    )(qf, sf)


def _sc_gather(table, idx_flat):
    bt, C = table.shape
    info = plsc.get_sparse_core_info()
    nw = info.num_cores * info.num_subcores
    chunk = 128
    nch = bt // (nw * chunk)
    idx3 = idx_flat.reshape(nw, nch, chunk)
    mesh = plsc.VectorSubcoreMesh(core_axis_name="c", subcore_axis_name="s")

    @functools.partial(
        pl.kernel,
        mesh=mesh,
        out_type=jax.ShapeDtypeStruct((nw, nch, chunk, C), jnp.float32),
        scratch_types=[
            pltpu.VMEM((nch, chunk), jnp.int32),
            pltpu.VMEM((nch, chunk, C), jnp.float32),
            pltpu.SemaphoreType.DMA,
        ],
    )
    def gather_k(table_hbm, idx_hbm, out_hbm, idx_v, rows_v, sem):
        wid = lax.axis_index("s") * info.num_cores + lax.axis_index("c")
        pltpu.sync_copy(idx_hbm.at[wid], idx_v)
        cps = [pltpu.async_copy(table_hbm.at[idx_v.at[j]], rows_v.at[j], sem)
               for j in range(nch)]
        for cp in cps:
            cp.wait()
        pltpu.sync_copy(rows_v, out_hbm.at[wid])

    out = gather_k(table, idx3)
    return out.reshape(bt, C)


def _fuse_body(val_ref, selt_ref, qf_ref, sf_ref, m_ref, w_ref, b_ref,
               out_ref):
    v = val_ref[0]
    e = jnp.exp(v - jnp.max(v))
    sw = e / jnp.sum(e)
    C = qf_ref.shape[1]
    sel = jnp.transpose(selt_ref[0, :, :C])
    fuse = sel * sw
    hybrid = jnp.concatenate([fuse, qf_ref[0]], axis=0)
    out = lax.dot_general(w_ref[...], hybrid, (((1,), (0,)), ((), ())),
                          preferred_element_type=jnp.float32)
    out = out + b_ref[...]
    out_ref[0] = out * m_ref[0] + sf_ref[0]


def _fuse(vals, selt, qf, sf, m3, W, b2):
    B, C, N = qf.shape
    return pl.pallas_call(
        _fuse_body,
        grid=(B,),
        in_specs=[
            pl.BlockSpec((1, 1, N), lambda b: (b, 0, 0)),
            pl.BlockSpec((1, N, 128), lambda b: (b, 0, 0)),
            pl.BlockSpec((1, C, N), lambda b: (b, 0, 0)),
            pl.BlockSpec((1, C, N), lambda b: (b, 0, 0)),
            pl.BlockSpec((1, 1, N), lambda b: (b, 0, 0)),
            pl.BlockSpec((C, 2 * C), lambda b: (0, 0)),
            pl.BlockSpec((C, 1), lambda b: (0, 0)),
        ],
        out_specs=pl.BlockSpec((1, C, N), lambda b: (b, 0, 0)),
        out_shape=jax.ShapeDtypeStruct((B, C, N), jnp.float32),
    )(vals, selt, qf, sf, m3, W, b2)


def kernel(x, mask, W, b):
    B, C, H, Wd = x.shape
    N = H * Wd
    h, w = mask.shape[2], mask.shape[3]
    ih = (jnp.arange(H) * h) // H
    iw = (jnp.arange(Wd) * w) // Wd
    m3 = mask[:, :, ih, :][:, :, :, iw].reshape(B, 1, N)
    x3 = x.reshape(B, C, N)

    qf, sf, sft = _prep(x3, m3)
    blk = 512
    vals3, idx3 = _top1(qf, sf, blk)
    selt = _sc_gather(sft.reshape(B * N, 128), idx3.reshape(B * N))
    vals = vals3.reshape(B, 1, N)
    out = _fuse(vals, selt.reshape(B, N, 128), qf, sf, m3, W,
                b.reshape(C, 1))
    return out.reshape(B, C, H, Wd)


# speedup vs baseline: 108.0076x; 1.1765x over previous
import functools

import jax
import jax.numpy as jnp
from jax import lax
from jax.experimental import pallas as pl
from jax.experimental.pallas import tpu as pltpu
from jax.experimental.pallas import tpu_sc as plsc


def _main_body(x_ref, m_ref, qf_ref, sf_ref, sft_ref, idx_ref, *,
               n_total, blk, nch):
    b = pl.program_id(0)
    i = pl.program_id(1)
    off = i * blk
    xb = x_ref[0]
    mb = m_ref[0]
    C = xb.shape[0]
    sf_full = xb * (1.0 - mb)
    xblk = x_ref[0, :, pl.ds(off, blk)]
    mblk = m_ref[0, :, pl.ds(off, blk)]
    qblk = xblk * mblk
    sfblk = xblk * (1.0 - mblk)
    qf_ref[0] = qblk
    sf_ref[0] = sfblk
    sft_ref[0, :, :C] = jnp.transpose(sfblk)
    sft_ref[0, :, C:] = jnp.zeros((blk, 128 - C), jnp.float32)
    s = lax.dot_general(qblk.astype(jnp.bfloat16),
                        sf_full.astype(jnp.bfloat16),
                        (((0,), (0,)), ((), ())),
                        preferred_element_type=jnp.float32)
    mx = jnp.max(s, axis=1, keepdims=True)
    col = lax.broadcasted_iota(jnp.int32, s.shape, 1)
    am = jnp.min(jnp.where(s >= mx, col, n_total), axis=1)
    idx_ref[...] = (am + b * n_total).reshape(1, nch, 128)


def _main(x3, m3, blk):
    B, C, N = x3.shape
    nb = N // blk
    nch = blk // 128
    body = functools.partial(_main_body, n_total=N, blk=blk, nch=nch)
    return pl.pallas_call(
        body,
        grid=(B, nb),
        in_specs=[
            pl.BlockSpec((1, C, N), lambda b, i: (b, 0, 0)),
            pl.BlockSpec((1, 1, N), lambda b, i: (b, 0, 0)),
        ],
        out_specs=[
            pl.BlockSpec((1, C, blk), lambda b, i: (b, 0, i)),
            pl.BlockSpec((1, C, blk), lambda b, i: (b, 0, i)),
            pl.BlockSpec((1, blk, 128), lambda b, i: (b, i, 0)),
            pl.BlockSpec((1, nch, 128), lambda b, i: (b * nb + i, 0, 0)),
        ],
        out_shape=[
            jax.ShapeDtypeStruct((B, C, N), jnp.float32),
            jax.ShapeDtypeStruct((B, C, N), jnp.float32),
            jax.ShapeDtypeStruct((B, N, 128), jnp.float32),
            jax.ShapeDtypeStruct((B * nb, nch, 128), jnp.int32),
        ],
    )(x3, m3)


def _sc_gather(table, idx3):
    bt, D = table.shape
    nw, nch, chunk = idx3.shape
    info = plsc.get_sparse_core_info()
    mesh = plsc.VectorSubcoreMesh(core_axis_name="c", subcore_axis_name="s")

    @functools.partial(
        pl.kernel,
        mesh=mesh,
        out_type=jax.ShapeDtypeStruct((nw, nch, chunk, D), jnp.float32),
        scratch_types=[
            pltpu.VMEM((nch, chunk), jnp.int32),
            pltpu.VMEM((nch, chunk, D), jnp.float32),
            pltpu.SemaphoreType.DMA,
        ],
    )
    def gather_k(table_hbm, idx_hbm, out_hbm, idx_v, rows_v, sem):
        wid = lax.axis_index("s") * info.num_cores + lax.axis_index("c")
        pltpu.sync_copy(idx_hbm.at[wid], idx_v)
        cps = [pltpu.async_copy(table_hbm.at[idx_v.at[j]], rows_v.at[j], sem)
               for j in range(nch)]
        for cp in cps:
            cp.wait()
        pltpu.sync_copy(rows_v, out_hbm.at[wid])

    return gather_k(table, idx3).reshape(bt, D)


def _fuse_body(selt_ref, qf_ref, sf_ref, m_ref, w_ref, b_ref, out_ref):
    C = qf_ref.shape[1]
    sel = jnp.transpose(selt_ref[0, :, :C])
    qfb = qf_ref[0]
    v = jnp.sum(qfb * sel, axis=0, keepdims=True)
    e = jnp.exp(v - jnp.max(v))
    sw = e / jnp.sum(e)
    fuse = sel * sw
    hybrid = jnp.concatenate([fuse, qfb], axis=0)
    out = lax.dot_general(w_ref[...], hybrid, (((1,), (0,)), ((), ())),
                          preferred_element_type=jnp.float32)
    out = out + b_ref[...]
    out_ref[0] = out * m_ref[0] + sf_ref[0]


def _fuse(selt, qf, sf, m3, W, b2):
    B, C, N = qf.shape
    return pl.pallas_call(
        _fuse_body,
        grid=(B,),
        in_specs=[
            pl.BlockSpec((1, N, 128), lambda b: (b, 0, 0)),
            pl.BlockSpec((1, C, N), lambda b: (b, 0, 0)),
            pl.BlockSpec((1, C, N), lambda b: (b, 0, 0)),
            pl.BlockSpec((1, 1, N), lambda b: (b, 0, 0)),
            pl.BlockSpec((C, 2 * C), lambda b: (0, 0)),
            pl.BlockSpec((C, 1), lambda b: (0, 0)),
        ],
        out_specs=pl.BlockSpec((1, C, N), lambda b: (b, 0, 0)),
        out_shape=jax.ShapeDtypeStruct((B, C, N), jnp.float32),
    )(selt, qf, sf, m3, W, b2)


def kernel(x, mask, W, b):
    B, C, H, Wd = x.shape
    N = H * Wd
    h, w = mask.shape[2], mask.shape[3]
    ih = (jnp.arange(H) * h) // H
    iw = (jnp.arange(Wd) * w) // Wd
    m3 = mask[:, :, ih, :][:, :, :, iw].reshape(B, 1, N)
    x3 = x.reshape(B, C, N)

    qf, sf, sft, idx3 = _main(x3, m3, 512)
    selt = _sc_gather(sft.reshape(B * N, 128), idx3)
    out = _fuse(selt.reshape(B, N, 128), qf, sf, m3, W, b.reshape(C, 1))
    return out.reshape(B, C, H, Wd)


# speedup vs baseline: 108.9515x; 1.0087x over previous
import functools

import jax
import jax.numpy as jnp
from jax import lax
from jax.experimental import pallas as pl
from jax.experimental.pallas import tpu as pltpu
from jax.experimental.pallas import tpu_sc as plsc


def _main_body(x_ref, m_ref, sft_ref, idx_ref, *, n_total, blk, nch):
    b = pl.program_id(0)
    i = pl.program_id(1)
    off = i * blk
    xb = x_ref[0]
    mb = m_ref[0]
    C = xb.shape[0]
    sf_full = xb * (1.0 - mb)
    xblk = x_ref[0, :, pl.ds(off, blk)]
    mblk = m_ref[0, :, pl.ds(off, blk)]
    qblk = xblk * mblk
    sfblk = xblk * (1.0 - mblk)
    eye = (lax.broadcasted_iota(jnp.int32, (C, C), 0) ==
           lax.broadcasted_iota(jnp.int32, (C, C), 1)).astype(jnp.float32)
    sft_ref[0, :, :C] = lax.dot_general(
        sfblk, eye, (((0,), (0,)), ((), ())),
        preferred_element_type=jnp.float32)
    s = lax.dot_general(qblk, sf_full,
                        (((0,), (0,)), ((), ())),
                        preferred_element_type=jnp.float32)
    mx = jnp.max(s, axis=1, keepdims=True)
    col = lax.broadcasted_iota(jnp.int32, s.shape, 1)
    am = jnp.min(jnp.where(s >= mx, col, n_total), axis=1)
    idx_ref[...] = (am + b * n_total).reshape(1, nch, 128)


def _main(x3, m3, blk):
    B, C, N = x3.shape
    nb = N // blk
    nch = blk // 128
    body = functools.partial(_main_body, n_total=N, blk=blk, nch=nch)
    return pl.pallas_call(
        body,
        grid=(B, nb),
        in_specs=[
            pl.BlockSpec((1, C, N), lambda b, i: (b, 0, 0)),
            pl.BlockSpec((1, 1, N), lambda b, i: (b, 0, 0)),
        ],
        out_specs=[
            pl.BlockSpec((1, blk, 128), lambda b, i: (b, i, 0)),
            pl.BlockSpec((1, nch, 128), lambda b, i: (b * nb + i, 0, 0)),
        ],
        out_shape=[
            jax.ShapeDtypeStruct((B, N, 128), jnp.float32),
            jax.ShapeDtypeStruct((B * nb, nch, 128), jnp.int32),
        ],
    )(x3, m3)


def _sc_gather(table, idx3):
    bt, D = table.shape
    nw, nch, chunk = idx3.shape
    info = plsc.get_sparse_core_info()
    mesh = plsc.VectorSubcoreMesh(core_axis_name="c", subcore_axis_name="s")

    @functools.partial(
        pl.kernel,
        mesh=mesh,
        out_type=jax.ShapeDtypeStruct((nw, nch, chunk, D), jnp.float32),
        scratch_types=[
            pltpu.VMEM((nch, chunk), jnp.int32),
            pltpu.VMEM((nch, chunk, D), jnp.float32),
            pltpu.SemaphoreType.DMA,
        ],
    )
    def gather_k(table_hbm, idx_hbm, out_hbm, idx_v, rows_v, sem):
        wid = lax.axis_index("s") * info.num_cores + lax.axis_index("c")
        pltpu.sync_copy(idx_hbm.at[wid], idx_v)
        cps = [pltpu.async_copy(table_hbm.at[idx_v.at[j]], rows_v.at[j], sem)
               for j in range(nch)]
        for cp in cps:
            cp.wait()
        pltpu.sync_copy(rows_v, out_hbm.at[wid])

    return gather_k(table, idx3).reshape(bt, D)


def _fuse_body(selt_ref, x_ref, m_ref, w_ref, b_ref, out_ref):
    xb = x_ref[0]
    mb = m_ref[0]
    C = xb.shape[0]
    qfb = xb * mb
    sfb = xb * (1.0 - mb)
    sel = jnp.transpose(selt_ref[0, :, :C])
    v = jnp.sum(qfb * sel, axis=0, keepdims=True)
    e = jnp.exp(v - jnp.max(v))
    sw = e / jnp.sum(e)
    fuse = sel * sw
    hybrid = jnp.concatenate([fuse, qfb], axis=0)
    out = lax.dot_general(w_ref[...], hybrid, (((1,), (0,)), ((), ())),
                          preferred_element_type=jnp.float32)
    out = out + b_ref[...]
    out_ref[0] = out * mb + sfb


def _fuse(selt, x3, m3, W, b2):
    B, C, N = x3.shape
    return pl.pallas_call(
        _fuse_body,
        grid=(B,),
        in_specs=[
            pl.BlockSpec((1, N, 128), lambda b: (b, 0, 0)),
            pl.BlockSpec((1, C, N), lambda b: (b, 0, 0)),
            pl.BlockSpec((1, 1, N), lambda b: (b, 0, 0)),
            pl.BlockSpec((C, 2 * C), lambda b: (0, 0)),
            pl.BlockSpec((C, 1), lambda b: (0, 0)),
        ],
        out_specs=pl.BlockSpec((1, C, N), lambda b: (b, 0, 0)),
        out_shape=jax.ShapeDtypeStruct((B, C, N), jnp.float32),
    )(selt, x3, m3, W, b2)


def kernel(x, mask, W, b):
    B, C, H, Wd = x.shape
    N = H * Wd
    h, w = mask.shape[2], mask.shape[3]
    ih = (jnp.arange(H) * h) // H
    iw = (jnp.arange(Wd) * w) // Wd
    m3 = mask[:, :, ih, :][:, :, :, iw].reshape(B, 1, N)
    x3 = x.reshape(B, C, N)

    sft, idx3 = _main(x3, m3, 512)
    selt = _sc_gather(sft.reshape(B * N, 128), idx3)
    out = _fuse(selt.reshape(B, N, 128), x3, m3, W, b.reshape(C, 1))
    return out.reshape(B, C, H, Wd)
